# baseline (speedup 1.0000x reference)
import functools

import jax
import jax.numpy as jnp
from jax.experimental import pallas as pl
from jax.experimental.pallas import tpu as pltpu

LAMB = 0.5
CHAMFER_W = 0.1

_B = 4
_P = 128
_ROWS = 392
_LANES = 128
_NPIX = _ROWS * _LANES
_NTOT = _B * _NPIX
_KB = _P // 8


def _body(lo_ref, hi_ref, tgt_ref, prd_ref, out_ref, acc_ref):
    n = pl.program_id(0)

    @pl.when(n == 0)
    def _init():
        acc_ref[0] = 0.0
        acc_ref[1] = 0.0
        acc_ref[2] = 0.0

    bc = 0.5 * (lo_ref[0] + hi_ref[0])
    bins = [bc[8 * k:8 * k + 8, :] for k in range(_KB)]

    big = jnp.float32(1e10)
    zrow = jnp.zeros((1, _LANES), jnp.float32)
    init_run = tuple(jnp.full((8, _LANES), big, jnp.float32) for _ in range(_KB))

    def row_step(r, carry):
        s1, s2, ysum, cnt, runs = carry
        tg = tgt_ref[0, pl.ds(r, 1), :]
        pr = prd_ref[0, pl.ds(r, 1), :]
        m = tg >= 0.001
        pix = jnp.where(m, tg, big)

        ds = [jnp.abs(pix - b) for b in bins]
        new_runs = tuple(jnp.minimum(ru, d) for ru, d in zip(runs, ds))
        t = ds[0]
        for d in ds[1:]:
            t = jnp.minimum(t, d)
        pmin = jnp.min(t, axis=0, keepdims=True)
        ysum = ysum + jnp.where(m, pmin * pmin, 0.0)
        cnt = cnt + jnp.where(m, 1.0, 0.0)

        diff = jnp.log(pr) - jnp.log(tg)
        s1 = s1 + diff
        s2 = s2 + diff * diff
        return s1, s2, ysum, cnt, new_runs

    s1, s2, ysum, cnt, runs = jax.lax.fori_loop(
        0, _ROWS, row_step, (zrow, zrow, zrow, zrow, init_run))

    acc_ref[0] = acc_ref[0] + jnp.sum(s1)
    acc_ref[1] = acc_ref[1] + jnp.sum(s2)

    cham_y = jnp.sum(ysum) / jnp.maximum(jnp.sum(cnt), 1.0)
    xs = jnp.float32(0.0)
    for ru in runs:
        bm = jnp.min(ru, axis=1)
        xs = xs + jnp.sum(bm * bm)
    cham_x = xs / _P
    acc_ref[2] = acc_ref[2] + cham_x + cham_y

    @pl.when(n == _B - 1)
    def _fin():
        mu1 = acc_ref[0] / _NTOT
        mu2 = acc_ref[1] / _NTOT
        depth = jnp.sqrt(mu2 - LAMB * mu1 * mu1) * 10.0
        total = depth + CHAMFER_W * (acc_ref[2] / _B)
        out_ref[...] = jnp.full((1, 1), total, jnp.float32)


@jax.jit
def _run(e_lo, e_hi, tgt, prd):
    out = pl.pallas_call(
        _body,
        grid=(_B,),
        in_specs=[
            pl.BlockSpec((1, _P, 1), lambda n: (n, 0, 0)),
            pl.BlockSpec((1, _P, 1), lambda n: (n, 0, 0)),
            pl.BlockSpec((1, _ROWS, _LANES), lambda n: (n, 0, 0)),
            pl.BlockSpec((1, _ROWS, _LANES), lambda n: (n, 0, 0)),
        ],
        out_specs=pl.BlockSpec((1, 1), lambda n: (0, 0)),
        out_shape=jax.ShapeDtypeStruct((1, 1), jnp.float32),
        scratch_shapes=[pltpu.SMEM((3,), jnp.float32)],
    )(e_lo, e_hi, tgt, prd)
    return out[0, 0]


def kernel(bin_edges, pred, target):
    e_lo = bin_edges[:, :-1].reshape(_B, _P, 1)
    e_hi = bin_edges[:, 1:].reshape(_B, _P, 1)
    tgt = target.reshape(_B, _ROWS, _LANES)
    prd = pred.reshape(_B, _ROWS, _LANES)
    return _run(e_lo, e_hi, tgt, prd)


# speedup vs baseline: 2.9328x; 2.9328x over previous
import functools

import jax
import jax.numpy as jnp
from jax.experimental import pallas as pl
from jax.experimental.pallas import tpu as pltpu

LAMB = 0.5
CHAMFER_W = 0.1

_B = 4
_P = 128
_ROWS = 392
_LANES = 128
_NPIX = _ROWS * _LANES
_NTOT = _B * _NPIX
_KB = _P // 8


def _body(lo_ref, hi_ref, tgt_ref, prd_ref, out_ref, acc_ref):
    n = pl.program_id(0)

    @pl.when(n == 0)
    def _init():
        acc_ref[0] = 0.0
        acc_ref[1] = 0.0
        acc_ref[2] = 0.0

    bc = 0.5 * (lo_ref[0] + hi_ref[0])
    bins = [bc[8 * k:8 * k + 8, :] for k in range(_KB)]

    big = jnp.float32(1e10)
    zrow = jnp.zeros((1, _LANES), jnp.float32)
    init_run = tuple(jnp.full((8, _LANES), big, jnp.float32) for _ in range(_KB))

    U = 4

    def row_step(i, carry):
        s1, s2, ysum, cnt, runs = carry
        r = i * U
        tg = tgt_ref[0, pl.ds(r, U), :]
        pr = prd_ref[0, pl.ds(r, U), :]
        runs = list(runs)
        for u in range(U):
            tgu = tg[u:u + 1, :]
            m = tgu >= 0.001
            pix = jnp.where(m, tgu, big)

            ds = [jnp.abs(pix - b) for b in bins]
            for k in range(_KB):
                runs[k] = jnp.minimum(runs[k], ds[k])
            t = ds[0]
            for d in ds[1:]:
                t = jnp.minimum(t, d)
            pmin = jnp.min(t, axis=0, keepdims=True)
            ysum = ysum + jnp.where(m, pmin * pmin, 0.0)
            cnt = cnt + jnp.where(m, 1.0, 0.0)

        diff = jnp.log(pr) - jnp.log(tg)
        s1 = s1 + jnp.sum(diff, axis=0, keepdims=True)
        s2 = s2 + jnp.sum(diff * diff, axis=0, keepdims=True)
        return s1, s2, ysum, cnt, tuple(runs)

    s1, s2, ysum, cnt, runs = jax.lax.fori_loop(
        0, _ROWS // U, row_step, (zrow, zrow, zrow, zrow, init_run))

    acc_ref[0] = acc_ref[0] + jnp.sum(s1)
    acc_ref[1] = acc_ref[1] + jnp.sum(s2)

    cham_y = jnp.sum(ysum) / jnp.maximum(jnp.sum(cnt), 1.0)
    xs = jnp.float32(0.0)
    for ru in runs:
        bm = jnp.min(ru, axis=1)
        xs = xs + jnp.sum(bm * bm)
    cham_x = xs / _P
    acc_ref[2] = acc_ref[2] + cham_x + cham_y

    @pl.when(n == _B - 1)
    def _fin():
        mu1 = acc_ref[0] / _NTOT
        mu2 = acc_ref[1] / _NTOT
        depth = jnp.sqrt(mu2 - LAMB * mu1 * mu1) * 10.0
        total = depth + CHAMFER_W * (acc_ref[2] / _B)
        out_ref[...] = jnp.full((1, 1), total, jnp.float32)


@jax.jit
def _run(e_lo, e_hi, tgt, prd):
    out = pl.pallas_call(
        _body,
        grid=(_B,),
        in_specs=[
            pl.BlockSpec((1, _P, 1), lambda n: (n, 0, 0)),
            pl.BlockSpec((1, _P, 1), lambda n: (n, 0, 0)),
            pl.BlockSpec((1, _ROWS, _LANES), lambda n: (n, 0, 0)),
            pl.BlockSpec((1, _ROWS, _LANES), lambda n: (n, 0, 0)),
        ],
        out_specs=pl.BlockSpec((1, 1), lambda n: (0, 0)),
        out_shape=jax.ShapeDtypeStruct((1, 1), jnp.float32),
        scratch_shapes=[pltpu.SMEM((3,), jnp.float32)],
    )(e_lo, e_hi, tgt, prd)
    return out[0, 0]


def kernel(bin_edges, pred, target):
    e_lo = bin_edges[:, :-1].reshape(_B, _P, 1)
    e_hi = bin_edges[:, 1:].reshape(_B, _P, 1)
    tgt = target.reshape(_B, _ROWS, _LANES)
    prd = pred.reshape(_B, _ROWS, _LANES)
    return _run(e_lo, e_hi, tgt, prd)


# speedup vs baseline: 4.0214x; 1.3712x over previous
import functools

import jax
import jax.numpy as jnp
from jax.experimental import pallas as pl
from jax.experimental.pallas import tpu as pltpu

LAMB = 0.5
CHAMFER_W = 0.1

_B = 4
_P = 128
_ROWS = 392
_LANES = 128
_NPIX = _ROWS * _LANES
_NTOT = _B * _NPIX
_KB = _P // 8


def _body(lo_ref, hi_ref, tgt_ref, prd_ref, out_ref, acc_ref):
    n = pl.program_id(0)

    @pl.when(n == 0)
    def _init():
        acc_ref[0] = 0.0
        acc_ref[1] = 0.0
        acc_ref[2] = 0.0

    bc = 0.5 * (lo_ref[0] + hi_ref[0])
    bins = [bc[8 * k:8 * k + 8, :] for k in range(_KB)]

    big = jnp.float32(1e10)
    zrow = jnp.zeros((1, _LANES), jnp.float32)
    init_run = tuple(jnp.full((8, _LANES), big, jnp.float32) for _ in range(_KB))

    U = 8

    def row_step(i, carry):
        s1, s2, ysum, cnt, runs = carry
        r = i * U
        tg = tgt_ref[0, pl.ds(r, U), :]
        pr = prd_ref[0, pl.ds(r, U), :]
        runs = list(runs)
        for u in range(U):
            tgu = tg[u:u + 1, :]
            m = tgu >= 0.001
            pix = jnp.where(m, tgu, big)

            ds = [jnp.abs(pix - b) for b in bins]
            for k in range(_KB):
                runs[k] = jnp.minimum(runs[k], ds[k])
            t = ds[0]
            for d in ds[1:]:
                t = jnp.minimum(t, d)
            pmin = jnp.min(t, axis=0, keepdims=True)
            ysum = ysum + jnp.where(m, pmin * pmin, 0.0)
            cnt = cnt + jnp.where(m, 1.0, 0.0)

        diff = jnp.log(pr) - jnp.log(tg)
        s1 = s1 + jnp.sum(diff, axis=0, keepdims=True)
        s2 = s2 + jnp.sum(diff * diff, axis=0, keepdims=True)
        return s1, s2, ysum, cnt, tuple(runs)

    s1, s2, ysum, cnt, runs = jax.lax.fori_loop(
        0, _ROWS // U, row_step, (zrow, zrow, zrow, zrow, init_run))

    acc_ref[0] = acc_ref[0] + jnp.sum(s1)
    acc_ref[1] = acc_ref[1] + jnp.sum(s2)

    cham_y = jnp.sum(ysum) / jnp.maximum(jnp.sum(cnt), 1.0)
    xs = jnp.float32(0.0)
    for ru in runs:
        bm = jnp.min(ru, axis=1)
        xs = xs + jnp.sum(bm * bm)
    cham_x = xs / _P
    acc_ref[2] = acc_ref[2] + cham_x + cham_y

    @pl.when(n == _B - 1)
    def _fin():
        mu1 = acc_ref[0] / _NTOT
        mu2 = acc_ref[1] / _NTOT
        depth = jnp.sqrt(mu2 - LAMB * mu1 * mu1) * 10.0
        total = depth + CHAMFER_W * (acc_ref[2] / _B)
        out_ref[...] = jnp.full((1, 1), total, jnp.float32)


@jax.jit
def _run(e_lo, e_hi, tgt, prd):
    out = pl.pallas_call(
        _body,
        grid=(_B,),
        in_specs=[
            pl.BlockSpec((1, _P, 1), lambda n: (n, 0, 0)),
            pl.BlockSpec((1, _P, 1), lambda n: (n, 0, 0)),
            pl.BlockSpec((1, _ROWS, _LANES), lambda n: (n, 0, 0)),
            pl.BlockSpec((1, _ROWS, _LANES), lambda n: (n, 0, 0)),
        ],
        out_specs=pl.BlockSpec((1, 1), lambda n: (0, 0)),
        out_shape=jax.ShapeDtypeStruct((1, 1), jnp.float32),
        scratch_shapes=[pltpu.SMEM((3,), jnp.float32)],
    )(e_lo, e_hi, tgt, prd)
    return out[0, 0]


def kernel(bin_edges, pred, target):
    e_lo = bin_edges[:, :-1].reshape(_B, _P, 1)
    e_hi = bin_edges[:, 1:].reshape(_B, _P, 1)
    tgt = target.reshape(_B, _ROWS, _LANES)
    prd = pred.reshape(_B, _ROWS, _LANES)
    return _run(e_lo, e_hi, tgt, prd)


# speedup vs baseline: 4.8539x; 1.2070x over previous
import functools

import jax
import jax.numpy as jnp
from jax.experimental import pallas as pl
from jax.experimental.pallas import tpu as pltpu

LAMB = 0.5
CHAMFER_W = 0.1

_B = 4
_P = 128
_ROWS = 392
_LANES = 128
_NPIX = _ROWS * _LANES
_NTOT = _B * _NPIX
_KB = _P // 8


def _body(lo_ref, hi_ref, tgt_ref, prd_ref, out_ref, acc_ref):
    n = pl.program_id(0)

    @pl.when(n == 0)
    def _init():
        acc_ref[0] = 0.0
        acc_ref[1] = 0.0
        acc_ref[2] = 0.0

    bc = 0.5 * (lo_ref[0] + hi_ref[0])
    bins = [bc[8 * k:8 * k + 8, :] for k in range(_KB)]

    big = jnp.float32(1e10)
    zrow = jnp.zeros((1, _LANES), jnp.float32)
    init_run = tuple(jnp.full((8, _LANES), big, jnp.float32) for _ in range(_KB))

    U = 14

    def row_step(i, carry):
        s1, s2, ysum, cnt, runs = carry
        r = i * U
        tg = tgt_ref[0, pl.ds(r, U), :]
        pr = prd_ref[0, pl.ds(r, U), :]
        runs = list(runs)
        for u in range(U):
            tgu = tg[u:u + 1, :]
            m = tgu >= 0.001
            pix = jnp.where(m, tgu, big)

            ds = [jnp.abs(pix - b) for b in bins]
            for k in range(_KB):
                runs[k] = jnp.minimum(runs[k], ds[k])
            t = ds[0]
            for d in ds[1:]:
                t = jnp.minimum(t, d)
            pmin = jnp.min(t, axis=0, keepdims=True)
            ysum = ysum + jnp.where(m, pmin * pmin, 0.0)
            cnt = cnt + jnp.where(m, 1.0, 0.0)

        diff = jnp.log(pr) - jnp.log(tg)
        s1 = s1 + jnp.sum(diff, axis=0, keepdims=True)
        s2 = s2 + jnp.sum(diff * diff, axis=0, keepdims=True)
        return s1, s2, ysum, cnt, tuple(runs)

    s1, s2, ysum, cnt, runs = jax.lax.fori_loop(
        0, _ROWS // U, row_step, (zrow, zrow, zrow, zrow, init_run))

    acc_ref[0] = acc_ref[0] + jnp.sum(s1)
    acc_ref[1] = acc_ref[1] + jnp.sum(s2)

    cham_y = jnp.sum(ysum) / jnp.maximum(jnp.sum(cnt), 1.0)
    xs = jnp.float32(0.0)
    for ru in runs:
        bm = jnp.min(ru, axis=1)
        xs = xs + jnp.sum(bm * bm)
    cham_x = xs / _P
    acc_ref[2] = acc_ref[2] + cham_x + cham_y

    @pl.when(n == _B - 1)
    def _fin():
        mu1 = acc_ref[0] / _NTOT
        mu2 = acc_ref[1] / _NTOT
        depth = jnp.sqrt(mu2 - LAMB * mu1 * mu1) * 10.0
        total = depth + CHAMFER_W * (acc_ref[2] / _B)
        out_ref[...] = jnp.full((1, 1), total, jnp.float32)


@jax.jit
def _run(e_lo, e_hi, tgt, prd):
    out = pl.pallas_call(
        _body,
        grid=(_B,),
        in_specs=[
            pl.BlockSpec((1, _P, 1), lambda n: (n, 0, 0)),
            pl.BlockSpec((1, _P, 1), lambda n: (n, 0, 0)),
            pl.BlockSpec((1, _ROWS, _LANES), lambda n: (n, 0, 0)),
            pl.BlockSpec((1, _ROWS, _LANES), lambda n: (n, 0, 0)),
        ],
        out_specs=pl.BlockSpec((1, 1), lambda n: (0, 0)),
        out_shape=jax.ShapeDtypeStruct((1, 1), jnp.float32),
        scratch_shapes=[pltpu.SMEM((3,), jnp.float32)],
    )(e_lo, e_hi, tgt, prd)
    return out[0, 0]


def kernel(bin_edges, pred, target):
    e_lo = bin_edges[:, :-1].reshape(_B, _P, 1)
    e_hi = bin_edges[:, 1:].reshape(_B, _P, 1)
    tgt = target.reshape(_B, _ROWS, _LANES)
    prd = pred.reshape(_B, _ROWS, _LANES)
    return _run(e_lo, e_hi, tgt, prd)


# speedup vs baseline: 5.2797x; 1.0877x over previous
import functools

import jax
import jax.numpy as jnp
from jax.experimental import pallas as pl
from jax.experimental.pallas import tpu as pltpu

LAMB = 0.5
CHAMFER_W = 0.1

_B = 4
_P = 128
_ROWS = 392
_LANES = 128
_NPIX = _ROWS * _LANES
_NTOT = _B * _NPIX
_KB = _P // 8


def _body(lo_ref, hi_ref, tgt_ref, prd_ref, out_ref, acc_ref):
    n = pl.program_id(0)

    @pl.when(n == 0)
    def _init():
        acc_ref[0] = 0.0
        acc_ref[1] = 0.0
        acc_ref[2] = 0.0

    bc = 0.5 * (lo_ref[0] + hi_ref[0])
    bins = [bc[8 * k:8 * k + 8, :] for k in range(_KB)]

    big = jnp.float32(1e10)
    zrow = jnp.zeros((1, _LANES), jnp.float32)
    init_run = tuple(jnp.full((8, _LANES), big, jnp.float32) for _ in range(_KB))

    U = 28

    def row_step(i, carry):
        s1, s2, ysum, cnt, runs = carry
        r = i * U
        tg = tgt_ref[0, pl.ds(r, U), :]
        pr = prd_ref[0, pl.ds(r, U), :]
        runs = list(runs)
        for u in range(U):
            tgu = tg[u:u + 1, :]
            m = tgu >= 0.001
            pix = jnp.where(m, tgu, big)

            ds = [jnp.abs(pix - b) for b in bins]
            for k in range(_KB):
                runs[k] = jnp.minimum(runs[k], ds[k])
            t = ds[0]
            for d in ds[1:]:
                t = jnp.minimum(t, d)
            pmin = jnp.min(t, axis=0, keepdims=True)
            ysum = ysum + jnp.where(m, pmin * pmin, 0.0)
            cnt = cnt + jnp.where(m, 1.0, 0.0)

        diff = jnp.log(pr) - jnp.log(tg)
        s1 = s1 + jnp.sum(diff, axis=0, keepdims=True)
        s2 = s2 + jnp.sum(diff * diff, axis=0, keepdims=True)
        return s1, s2, ysum, cnt, tuple(runs)

    s1, s2, ysum, cnt, runs = jax.lax.fori_loop(
        0, _ROWS // U, row_step, (zrow, zrow, zrow, zrow, init_run))

    acc_ref[0] = acc_ref[0] + jnp.sum(s1)
    acc_ref[1] = acc_ref[1] + jnp.sum(s2)

    cham_y = jnp.sum(ysum) / jnp.maximum(jnp.sum(cnt), 1.0)
    xs = jnp.float32(0.0)
    for ru in runs:
        bm = jnp.min(ru, axis=1)
        xs = xs + jnp.sum(bm * bm)
    cham_x = xs / _P
    acc_ref[2] = acc_ref[2] + cham_x + cham_y

    @pl.when(n == _B - 1)
    def _fin():
        mu1 = acc_ref[0] / _NTOT
        mu2 = acc_ref[1] / _NTOT
        depth = jnp.sqrt(mu2 - LAMB * mu1 * mu1) * 10.0
        total = depth + CHAMFER_W * (acc_ref[2] / _B)
        out_ref[...] = jnp.full((1, 1), total, jnp.float32)


@jax.jit
def _run(e_lo, e_hi, tgt, prd):
    out = pl.pallas_call(
        _body,
        grid=(_B,),
        in_specs=[
            pl.BlockSpec((1, _P, 1), lambda n: (n, 0, 0)),
            pl.BlockSpec((1, _P, 1), lambda n: (n, 0, 0)),
            pl.BlockSpec((1, _ROWS, _LANES), lambda n: (n, 0, 0)),
            pl.BlockSpec((1, _ROWS, _LANES), lambda n: (n, 0, 0)),
        ],
        out_specs=pl.BlockSpec((1, 1), lambda n: (0, 0)),
        out_shape=jax.ShapeDtypeStruct((1, 1), jnp.float32),
        scratch_shapes=[pltpu.SMEM((3,), jnp.float32)],
    )(e_lo, e_hi, tgt, prd)
    return out[0, 0]


def kernel(bin_edges, pred, target):
    e_lo = bin_edges[:, :-1].reshape(_B, _P, 1)
    e_hi = bin_edges[:, 1:].reshape(_B, _P, 1)
    tgt = target.reshape(_B, _ROWS, _LANES)
    prd = pred.reshape(_B, _ROWS, _LANES)
    return _run(e_lo, e_hi, tgt, prd)


# speedup vs baseline: 5.7252x; 1.0844x over previous
import functools

import jax
import jax.numpy as jnp
from jax.experimental import pallas as pl
from jax.experimental.pallas import tpu as pltpu

LAMB = 0.5
CHAMFER_W = 0.1

_B = 4
_P = 128
_ROWS = 392
_LANES = 128
_NPIX = _ROWS * _LANES
_NTOT = _B * _NPIX
_KB = _P // 8


def _body(lo_ref, hi_ref, tgt_ref, prd_ref, out_ref, acc_ref):
    n = pl.program_id(0)

    @pl.when(n == 0)
    def _init():
        acc_ref[0] = 0.0
        acc_ref[1] = 0.0
        acc_ref[2] = 0.0

    bc = 0.5 * (lo_ref[0] + hi_ref[0])
    bins = [bc[8 * k:8 * k + 8, :] for k in range(_KB)]

    big = jnp.float32(1e10)
    zrow = jnp.zeros((1, _LANES), jnp.float32)
    init_run = tuple(jnp.full((8, _LANES), big, jnp.float32) for _ in range(_KB))

    U = 56

    def row_step(i, carry):
        s1, s2, ysum, cnt, runs = carry
        r = i * U
        tg = tgt_ref[0, pl.ds(r, U), :]
        pr = prd_ref[0, pl.ds(r, U), :]
        runs = list(runs)
        for u in range(U):
            tgu = tg[u:u + 1, :]
            m = tgu >= 0.001
            pix = jnp.where(m, tgu, big)

            ds = [jnp.abs(pix - b) for b in bins]
            for k in range(_KB):
                runs[k] = jnp.minimum(runs[k], ds[k])
            t = ds[0]
            for d in ds[1:]:
                t = jnp.minimum(t, d)
            pmin = jnp.min(t, axis=0, keepdims=True)
            ysum = ysum + jnp.where(m, pmin * pmin, 0.0)
            cnt = cnt + jnp.where(m, 1.0, 0.0)

        diff = jnp.log(pr) - jnp.log(tg)
        s1 = s1 + jnp.sum(diff, axis=0, keepdims=True)
        s2 = s2 + jnp.sum(diff * diff, axis=0, keepdims=True)
        return s1, s2, ysum, cnt, tuple(runs)

    s1, s2, ysum, cnt, runs = jax.lax.fori_loop(
        0, _ROWS // U, row_step, (zrow, zrow, zrow, zrow, init_run))

    acc_ref[0] = acc_ref[0] + jnp.sum(s1)
    acc_ref[1] = acc_ref[1] + jnp.sum(s2)

    cham_y = jnp.sum(ysum) / jnp.maximum(jnp.sum(cnt), 1.0)
    xs = jnp.float32(0.0)
    for ru in runs:
        bm = jnp.min(ru, axis=1)
        xs = xs + jnp.sum(bm * bm)
    cham_x = xs / _P
    acc_ref[2] = acc_ref[2] + cham_x + cham_y

    @pl.when(n == _B - 1)
    def _fin():
        mu1 = acc_ref[0] / _NTOT
        mu2 = acc_ref[1] / _NTOT
        depth = jnp.sqrt(mu2 - LAMB * mu1 * mu1) * 10.0
        total = depth + CHAMFER_W * (acc_ref[2] / _B)
        out_ref[...] = jnp.full((1, 1), total, jnp.float32)


@jax.jit
def _run(e_lo, e_hi, tgt, prd):
    out = pl.pallas_call(
        _body,
        grid=(_B,),
        in_specs=[
            pl.BlockSpec((1, _P, 1), lambda n: (n, 0, 0)),
            pl.BlockSpec((1, _P, 1), lambda n: (n, 0, 0)),
            pl.BlockSpec((1, _ROWS, _LANES), lambda n: (n, 0, 0)),
            pl.BlockSpec((1, _ROWS, _LANES), lambda n: (n, 0, 0)),
        ],
        out_specs=pl.BlockSpec((1, 1), lambda n: (0, 0)),
        out_shape=jax.ShapeDtypeStruct((1, 1), jnp.float32),
        scratch_shapes=[pltpu.SMEM((3,), jnp.float32)],
    )(e_lo, e_hi, tgt, prd)
    return out[0, 0]


def kernel(bin_edges, pred, target):
    e_lo = bin_edges[:, :-1].reshape(_B, _P, 1)
    e_hi = bin_edges[:, 1:].reshape(_B, _P, 1)
    tgt = target.reshape(_B, _ROWS, _LANES)
    prd = pred.reshape(_B, _ROWS, _LANES)
    return _run(e_lo, e_hi, tgt, prd)
